# baseline (speedup 1.0000x reference)
import functools
import jax
import jax.numpy as jnp
from jax import lax
from jax.experimental import pallas as pl
from jax.experimental.pallas import tpu as pltpu
from jax.experimental.pallas import tpu_sc as plsc

D_MODEL = 64
SCALE = 8.0
NUM_CORES = 2
NUM_SUBCORES = 16
NUM_WORKERS = NUM_CORES * NUM_SUBCORES
CHUNK = 128


@functools.partial(jax.jit, static_argnames=("n_idx",))
def _emb_lookup(x_flat, lut_padded, n_idx):
    per_worker = n_idx // NUM_WORKERS
    n_chunks = per_worker // CHUNK
    mesh = plsc.VectorSubcoreMesh(core_axis_name="c", subcore_axis_name="s")

    @functools.partial(
        pl.kernel,
        mesh=mesh,
        out_type=jax.ShapeDtypeStruct((n_idx, D_MODEL), jnp.float32),
        scratch_types=[
            pltpu.VMEM((4, CHUNK), jnp.int32),
            pltpu.VMEM((2, CHUNK, 128), jnp.float32),
            pltpu.VMEM((2, CHUNK, D_MODEL), jnp.float32),
            pltpu.SemaphoreType.DMA,
            pltpu.SemaphoreType.DMA,
            pltpu.SemaphoreType.DMA,
            pltpu.SemaphoreType.DMA,
            pltpu.SemaphoreType.DMA,
            pltpu.SemaphoreType.DMA,
            pltpu.SemaphoreType.DMA,
            pltpu.SemaphoreType.DMA,
        ],
        compiler_params=pltpu.CompilerParams(use_tc_tiling_on_sc=True),
    )
    def body(x_hbm, lut_hbm, out_hbm, idxb, gbuf, sbuf,
             i0, i1, i2, i3, g0, g1, o0, o1):
        isems = (i0, i1, i2, i3)
        gsems = (g0, g1)
        osems = (o0, o1)
        wid = lax.axis_index("s") * NUM_CORES + lax.axis_index("c")
        base = wid * per_worker

        def idx_copy(g, b4):
            return pltpu.make_async_copy(
                x_hbm.at[pl.ds(base + g * CHUNK, CHUNK)], idxb.at[b4],
                isems[b4],
            )

        def gather(b, b4):
            return pltpu.make_async_copy(
                lut_hbm.at[idxb.at[b4]], gbuf.at[b], gsems[b]
            )

        def out_copy(g, b):
            return pltpu.make_async_copy(
                sbuf.at[b],
                out_hbm.at[pl.ds(base + g * CHUNK, CHUNK)],
                osems[b],
            )

        def scale(br):
            def scale_row(r, c):
                for m in range(D_MODEL // 16):
                    sbuf[br, r, pl.ds(16 * m, 16)] = (
                        gbuf[br, r, pl.ds(16 * m, 16)] * SCALE
                    )
                return c

            lax.fori_loop(0, CHUNK, scale_row, 0, unroll=4)

        idx_copy(0, 0).start()
        idx_copy(1, 1).start()

        def step(s4, carry):
            for b4 in range(4):
                g = s4 * 4 + b4
                b = b4 % 2
                idx_copy(g, b4).wait()

                @pl.when(g > 1)
                def _():
                    out_copy(g - 2, b).wait()

                gather(b, b4).start()

                @pl.when(g + 2 < n_chunks)
                def _():
                    idx_copy(g + 2, (b4 + 2) % 4).start()

                bp = 1 - b

                @pl.when(g > 0)
                def _():
                    gather(bp, (b4 + 3) % 4).wait()
                    scale(bp)
                    out_copy(g - 1, bp).start()

            return carry

        lax.fori_loop(0, n_chunks // 4, step, 0)

        bl = (n_chunks - 1) % 2
        gather(bl, (n_chunks - 1) % 4).wait()
        scale(bl)
        out_copy(n_chunks - 1, bl).start()
        out_copy(n_chunks - 2, 1 - bl).wait()
        out_copy(n_chunks - 1, bl).wait()

    return body(x_flat, lut_padded)


def kernel(x, lut):
    n_idx = x.shape[0] * x.shape[1]
    x_flat = x.reshape(n_idx)
    lut_padded = jnp.pad(lut, ((0, 0), (0, 128 - D_MODEL)))
    out = _emb_lookup(x_flat, lut_padded, n_idx)
    return out.reshape(x.shape[0], x.shape[1], D_MODEL)


# speedup vs baseline: 1.1607x; 1.1607x over previous
import functools
import jax
import jax.numpy as jnp
from jax import lax
from jax.experimental import pallas as pl
from jax.experimental.pallas import tpu as pltpu
from jax.experimental.pallas import tpu_sc as plsc

D_MODEL = 64
SCALE = 8.0
NUM_CORES = 2
NUM_SUBCORES = 16
NUM_WORKERS = NUM_CORES * NUM_SUBCORES
CHUNK = 128


@functools.partial(jax.jit, static_argnames=("n_idx",))
def _emb_lookup(x_flat, lut_padded, n_idx):
    per_worker = n_idx // NUM_WORKERS
    n_chunks = per_worker // CHUNK
    mesh = plsc.VectorSubcoreMesh(core_axis_name="c", subcore_axis_name="s")

    @functools.partial(
        pl.kernel,
        mesh=mesh,
        out_type=jax.ShapeDtypeStruct((n_idx, D_MODEL), jnp.float32),
        scratch_types=[
            pltpu.VMEM((4, CHUNK), jnp.int32),
            pltpu.VMEM((2, CHUNK, 128), jnp.float32),
            pltpu.VMEM((2, CHUNK, D_MODEL), jnp.float32),
            pltpu.SemaphoreType.DMA,
            pltpu.SemaphoreType.DMA,
            pltpu.SemaphoreType.DMA,
            pltpu.SemaphoreType.DMA,
            pltpu.SemaphoreType.DMA,
            pltpu.SemaphoreType.DMA,
            pltpu.SemaphoreType.DMA,
            pltpu.SemaphoreType.DMA,
        ],
        compiler_params=pltpu.CompilerParams(use_tc_tiling_on_sc=True),
    )
    def body(x_hbm, lut_hbm, out_hbm, idxb, gbuf, sbuf,
             i0, i1, i2, i3, g0, g1, o0, o1):
        isems = (i0, i1, i2, i3)
        gsems = (g0, g1)
        osems = (o0, o1)
        wid = lax.axis_index("s") * NUM_CORES + lax.axis_index("c")
        base = wid * per_worker

        def idx_copy(g, b4):
            return pltpu.make_async_copy(
                x_hbm.at[pl.ds(base + g * CHUNK, CHUNK)], idxb.at[b4],
                isems[b4],
            )

        def gather(b, b4):
            return pltpu.make_async_copy(
                lut_hbm.at[idxb.at[b4]], gbuf.at[b], gsems[b]
            )

        def out_copy(g, b):
            return pltpu.make_async_copy(
                sbuf.at[b],
                out_hbm.at[pl.ds(base + g * CHUNK, CHUNK)],
                osems[b],
            )

        def scale(br):
            @plsc.parallel_loop(0, CHUNK, 1, unroll=4)
            def _scale_row(r):
                for m in range(D_MODEL // 16):
                    sbuf[br, r, pl.ds(16 * m, 16)] = (
                        gbuf[br, r, pl.ds(16 * m, 16)] * SCALE
                    )

        idx_copy(0, 0).start()
        idx_copy(1, 1).start()

        def step(s4, carry):
            for b4 in range(4):
                g = s4 * 4 + b4
                b = b4 % 2
                idx_copy(g, b4).wait()

                @pl.when(g > 1)
                def _():
                    out_copy(g - 2, b).wait()

                gather(b, b4).start()

                @pl.when(g + 2 < n_chunks)
                def _():
                    idx_copy(g + 2, (b4 + 2) % 4).start()

                bp = 1 - b

                @pl.when(g > 0)
                def _():
                    gather(bp, (b4 + 3) % 4).wait()
                    scale(bp)
                    out_copy(g - 1, bp).start()

            return carry

        lax.fori_loop(0, n_chunks // 4, step, 0)

        bl = (n_chunks - 1) % 2
        gather(bl, (n_chunks - 1) % 4).wait()
        scale(bl)
        out_copy(n_chunks - 1, bl).start()
        out_copy(n_chunks - 2, 1 - bl).wait()
        out_copy(n_chunks - 1, bl).wait()

    return body(x_flat, lut_padded)


def kernel(x, lut):
    n_idx = x.shape[0] * x.shape[1]
    x_flat = x.reshape(n_idx)
    lut_padded = jnp.pad(lut, ((0, 0), (0, 128 - D_MODEL)))
    out = _emb_lookup(x_flat, lut_padded, n_idx)
    return out.reshape(x.shape[0], x.shape[1], D_MODEL)
